# baseline (speedup 1.0000x reference)
import functools
import jax
import jax.numpy as jnp
from jax import lax
from jax.experimental import pallas as pl
from jax.experimental.pallas import tpu as pltpu

B, Q, C = 16, 20000, 92
QB = 2000
NQ = Q // QB


def _body(logits_ref, boxes_ref, scale_ref, scores_ref, labels_ref, oboxes_ref):
    x = logits_ref[0]
    m_all = jnp.max(x, axis=-1, keepdims=True)
    s = jnp.sum(jnp.exp(x - m_all), axis=-1, keepdims=True)
    col = lax.broadcasted_iota(jnp.int32, (QB, C), 1)
    xm = jnp.where(col < C - 1, x, -jnp.inf)
    m91 = jnp.max(xm, axis=-1, keepdims=True)
    scores_ref[0] = jnp.exp(m91 - m_all) / s
    labels_ref[0] = jnp.min(jnp.where(xm == m91, col, C), axis=-1, keepdims=True)

    bx = boxes_ref[0]
    xc = bx[:, 0:1]
    yc = bx[:, 1:2]
    w = bx[:, 2:3]
    h = bx[:, 3:4]
    xyxy = jnp.concatenate(
        [xc - 0.5 * w, yc - 0.5 * h, xc + 0.5 * w, yc + 0.5 * h], axis=-1
    )
    oboxes_ref[0] = xyxy * scale_ref[0]


@jax.jit
def _run(pred_logits, pred_boxes, scale):
    return pl.pallas_call(
        _body,
        grid=(B, NQ),
        in_specs=[
            pl.BlockSpec((1, QB, C), lambda b, q: (b, q, 0)),
            pl.BlockSpec((1, QB, 4), lambda b, q: (b, q, 0)),
            pl.BlockSpec((1, 1, 4), lambda b, q: (b, 0, 0)),
        ],
        out_specs=[
            pl.BlockSpec((1, QB, 1), lambda b, q: (b, q, 0)),
            pl.BlockSpec((1, QB, 1), lambda b, q: (b, q, 0)),
            pl.BlockSpec((1, QB, 4), lambda b, q: (b, q, 0)),
        ],
        out_shape=[
            jax.ShapeDtypeStruct((B, Q, 1), jnp.float32),
            jax.ShapeDtypeStruct((B, Q, 1), jnp.int32),
            jax.ShapeDtypeStruct((B, Q, 4), jnp.float32),
        ],
    )(pred_logits, pred_boxes, scale)


def kernel(pred_logits, pred_boxes, target_sizes):
    ts = target_sizes.astype(jnp.float32)
    img_h = ts[:, 0]
    img_w = ts[:, 1]
    scale = jnp.stack([img_w, img_h, img_w, img_h], axis=1)[:, None, :]
    scores, labels, boxes = _run(pred_logits, pred_boxes, scale)
    return scores[..., 0], labels[..., 0], boxes
